# baseline (speedup 1.0000x reference)
import functools

import jax
import jax.numpy as jnp
from jax import lax
from jax.experimental import pallas as pl
from jax.experimental.pallas import tpu as pltpu
from jax.experimental.pallas import tpu_sc as plsc

N_NODES = 50000
N_ELEM = 800000

NC, NS, LANES = 2, 16, 16
NW = NC * NS
NP = 50176
EPW = N_ELEM // NW
K = 1000
KB = 1008
NSTEP = KB // 16
NCHUNK = EPW // K
RPT = NP // NS
ZR = RPT // 8
NPW = NP // NW
F32 = jnp.float32
I32 = jnp.int32

_mesh = plsc.VectorSubcoreMesh(core_axis_name="c", subcore_axis_name="s")
_params = pltpu.CompilerParams(needs_layout_passes=False,
                               use_tc_tiling_on_sc=False)


def _rsqrt(x):
    x = jnp.maximum(x, 1e-16)
    i = plsc.bitcast(x, I32)
    i = jnp.int32(0x5F3759DF) - (i >> 1)
    y = plsc.bitcast(i, F32)
    for _ in range(2):
        y = y * (1.5 - 0.5 * x * y * y)
    return y


def _iota16():
    return lax.iota(I32, 16)


def _col(c):
    return jnp.full((16,), c, I32)


@functools.partial(
    pl.kernel,
    out_type=[
        jax.ShapeDtypeStruct((2 * NP, 8), F32),
        jax.ShapeDtypeStruct((2 * NW * 16,), F32),
    ],
    mesh=_mesh,
    compiler_params=_params,
    scratch_types=[
        pltpu.VMEM((2, 2 * K), I32),
        pltpu.VMEM((3, 2, K), I32),
        pltpu.VMEM((2, KB, 8), F32),
        pltpu.VMEM((2, KB, 8), F32),
        pltpu.VMEM((2, KB, 8), F32),
        pltpu.VMEM((2, KB, 8), F32),
        pltpu.VMEM((2, 3 * KB), F32),
        pltpu.VMEM((2, 4, KB), F32),
        pltpu.VMEM((ZR, 8), F32),
        pltpu.VMEM((16,), F32),
        pltpu.VMEM((16,), F32),
        pltpu.VMEM_SHARED((NP, 8), F32),
        [pltpu.SemaphoreType.DMA] * 2,
        [pltpu.SemaphoreType.DMA] * 2,
        [pltpu.SemaphoreType.DMA] * 2,
        [pltpu.SemaphoreType.DMA] * 2,
        [pltpu.SemaphoreType.DMA] * 2,
        [pltpu.SemaphoreType.DMA] * 2,
        [pltpu.SemaphoreType.DMA] * 2,
    ],
)
def _k1(conn, dirs, lh, eh, ah, ih, pred8,
        acc_out, sums_out,
        crbuf, idxb, rows_i, rows_j, srow_i, srow_j, xbuf, pbuf,
        zbuf, seA, seI, accS,
        sem_c, sem_d, sem_p, sem_gi, sem_gj, sem_si, sem_sj):
    c = lax.axis_index("c")
    s = lax.axis_index("s")
    wid = s * NC + c

    def _zb1(t, carry):
        e = t * 16 + _iota16()
        z = jnp.zeros((16,), F32)
        plsc.store_scatter(zbuf, [e >> 3, e & 7], z)
        return carry

    lax.fori_loop(0, (ZR * 8) // 16, _zb1, 0)
    for q in range(8):
        pltpu.sync_copy(zbuf, accS.at[pl.ds(s * RPT + q * ZR, ZR)])

    def _zpad(t, carry):
        e = t * 16 + _iota16()
        z = jnp.zeros((16,), F32)
        for b in range(2):
            plsc.store_scatter(srow_i.at[b], [e, _col(6)], z)
            plsc.store_scatter(srow_i.at[b], [e, _col(7)], z)
            plsc.store_scatter(srow_j.at[b], [e, _col(6)], z)
            plsc.store_scatter(srow_j.at[b], [e, _col(7)], z)
        return carry

    lax.fori_loop(0, NSTEP, _zpad, 0)
    plsc.subcore_barrier()

    ebase = wid * EPW

    def _issue_in(g):
        base = ebase + g * K
        dc = pltpu.async_copy(conn.at[pl.ds(base * 2, 2 * K)],
                              crbuf.at[g % 2], sem_c[g % 2])
        dd = pltpu.async_copy(dirs.at[pl.ds(base * 3, 3 * K)],
                              xbuf.at[g % 2].at[pl.ds(0, 3 * K)], sem_d[g % 2])
        dl = pltpu.async_copy(lh.at[pl.ds(base, K)],
                              pbuf.at[g % 2].at[0].at[pl.ds(0, K)], sem_p[g % 2])
        de = pltpu.async_copy(eh.at[pl.ds(base, K)],
                              pbuf.at[g % 2].at[1].at[pl.ds(0, K)], sem_p[g % 2])
        da = pltpu.async_copy(ah.at[pl.ds(base, K)],
                              pbuf.at[g % 2].at[2].at[pl.ds(0, K)], sem_p[g % 2])
        di = pltpu.async_copy(ih.at[pl.ds(base, K)],
                              pbuf.at[g % 2].at[3].at[pl.ds(0, K)], sem_p[g % 2])
        return dc, (dd, dl, de, da, di)

    def _build_idx(g):
        cr = crbuf.at[g % 2]
        ib = idxb.at[g % 3]

        def _bi(t, carry):
            e = t * 16 + _iota16()
            m = e < K
            e2 = e * 2
            vi = plsc.load_gather(cr, [e2])
            vj = plsc.load_gather(cr, [e2 + 1])
            plsc.store_scatter(ib, [_col(0), e], vi, mask=m)
            plsc.store_scatter(ib, [_col(1), e], vj, mask=m)
            return carry

        lax.fori_loop(0, NSTEP, _bi, 0)

    def _issue_gather(g):
        ib = idxb.at[g % 3]
        gi = pltpu.async_copy(pred8.at[ib.at[0]],
                              rows_i.at[g % 2].at[pl.ds(0, K)], sem_gi[g % 2])
        gj = pltpu.async_copy(pred8.at[ib.at[1]],
                              rows_j.at[g % 2].at[pl.ds(0, K)], sem_gj[g % 2])
        return gi, gj

    def _issue_scatter(g):
        ib = idxb.at[g % 3]
        di = pltpu.async_copy(srow_i.at[g % 2].at[pl.ds(0, K)],
                              accS.at[ib.at[0]], sem_si[g % 2], add=True)
        dj = pltpu.async_copy(srow_j.at[g % 2].at[pl.ds(0, K)],
                              accS.at[ib.at[1]], sem_sj[g % 2], add=True)
        return di, dj

    def _compute(g, carry):
        sa, si = carry
        xb = xbuf.at[g % 2]
        pb = pbuf.at[g % 2]
        ri = rows_i.at[g % 2]
        rj = rows_j.at[g % 2]
        swi = srow_i.at[g % 2]
        swj = srow_j.at[g % 2]

        def _step(t, carry2):
            sa2, si2 = carry2
            o = t * 16
            e = o + _iota16()
            e3 = e * 3
            ax = plsc.load_gather(xb, [e3])
            ay = plsc.load_gather(xb, [e3 + 1])
            az = plsc.load_gather(xb, [e3 + 2])
            lv = pb[0, pl.ds(o, 16)]
            ev = pb[1, pl.ds(o, 16)]
            av = pb[2, pl.ds(o, 16)]
            iv = pb[3, pl.ds(o, 16)]
            par = jnp.abs(ay) > 0.99
            zpx = jnp.where(par, ay, -az)
            zpy = jnp.where(par, -ax, jnp.zeros((16,), F32))
            zpz = jnp.where(par, jnp.zeros((16,), F32), ax)
            rs = _rsqrt(zpx * zpx + zpy * zpy + zpz * zpz)
            zx_, zy_, zz_ = zpx * rs, zpy * rs, zpz * rs
            ypx = zy_ * az - zz_ * ay
            ypy = zz_ * ax - zx_ * az
            ypz = zx_ * ay - zy_ * ax
            rs2 = _rsqrt(ypx * ypx + ypy * ypy + ypz * ypz)
            yx_, yy_, yz_ = ypx * rs2, ypy * rs2, ypz * rs2
            uix = plsc.load_gather(ri, [e, _col(0)])
            uiy = plsc.load_gather(ri, [e, _col(1)])
            uiz = plsc.load_gather(ri, [e, _col(2)])
            tix = plsc.load_gather(ri, [e, _col(3)])
            tiy = plsc.load_gather(ri, [e, _col(4)])
            tiz = plsc.load_gather(ri, [e, _col(5)])
            ujx = plsc.load_gather(rj, [e, _col(0)])
            ujy = plsc.load_gather(rj, [e, _col(1)])
            ujz = plsc.load_gather(rj, [e, _col(2)])
            tjx = plsc.load_gather(rj, [e, _col(3)])
            tjy = plsc.load_gather(rj, [e, _col(4)])
            tjz = plsc.load_gather(rj, [e, _col(5)])
            dux, duy, duz = ujx - uix, ujy - uiy, ujz - uiz
            irl = _rsqrt(lv)
            invl = irl * irl
            ei = ev * iv
            ei_l = ei * invl
            ei_l2 = ei_l * invl
            ei_l3 = ei_l2 * invl
            axial = dux * ax + duy * ay + duz * az
            na = ev * av * invl * axial
            dwz = dux * zx_ + duy * zy_ + duz * zz_
            thyi = tix * yx_ + tiy * yy_ + tiz * yz_
            thyj = tjx * yx_ + tjy * yy_ + tjz * yz_
            vz = 12.0 * ei_l3 * dwz - 6.0 * ei_l2 * (thyi + thyj)
            myi = 6.0 * ei_l2 * dwz - ei_l * (4.0 * thyi + 2.0 * thyj)
            myj = 6.0 * ei_l2 * dwz - ei_l * (2.0 * thyi + 4.0 * thyj)
            dwy = dux * yx_ + duy * yy_ + duz * yz_
            thzi = tix * zx_ + tiy * zy_ + tiz * zz_
            thzj = tjx * zx_ + tjy * zy_ + tjz * zz_
            vy = 12.0 * ei_l3 * dwy + 6.0 * ei_l2 * (thzi + thzj)
            mzi = -6.0 * ei_l2 * dwy - ei_l * (4.0 * thzi + 2.0 * thzj)
            mzj = -6.0 * ei_l2 * dwy - ei_l * (2.0 * thzi + 4.0 * thzj)
            fx = na * ax + vz * zx_ + vy * yx_
            fy = na * ay + vz * zy_ + vy * yy_
            fz = na * az + vz * zz_ + vy * yz_
            plsc.store_scatter(swi, [e, _col(0)], fx)
            plsc.store_scatter(swi, [e, _col(1)], fy)
            plsc.store_scatter(swi, [e, _col(2)], fz)
            plsc.store_scatter(swi, [e, _col(3)], myi * yx_ + mzi * zx_)
            plsc.store_scatter(swi, [e, _col(4)], myi * yy_ + mzi * zy_)
            plsc.store_scatter(swi, [e, _col(5)], myi * yz_ + mzi * zz_)
            plsc.store_scatter(swj, [e, _col(0)], -fx)
            plsc.store_scatter(swj, [e, _col(1)], -fy)
            plsc.store_scatter(swj, [e, _col(2)], -fz)
            plsc.store_scatter(swj, [e, _col(3)], myj * yx_ + mzj * zx_)
            plsc.store_scatter(swj, [e, _col(4)], myj * yy_ + mzj * zy_)
            plsc.store_scatter(swj, [e, _col(5)], myj * yz_ + mzj * zz_)
            valid = jnp.where(e < K, jnp.ones((16,), F32), jnp.zeros((16,), F32))
            return sa2 + valid * (ev * av), si2 + valid * ei_l

        return lax.fori_loop(0, NSTEP, _step, (sa, si))

    sa = jnp.zeros((16,), F32)
    si = jnp.zeros((16,), F32)
    dcs = [None, None]
    dis = [None, None]
    dcs[0], dis[0] = _issue_in(0)
    dcs[1], dis[1] = _issue_in(1)
    dcs[0].wait()
    _build_idx(0)
    gs = {0: _issue_gather(0)}
    scts = {}
    for g in range(NCHUNK):
        if g + 1 < NCHUNK:
            dcs[(g + 1) % 2].wait()
            _build_idx(g + 1)
            gs[g + 1] = _issue_gather(g + 1)
        for d in dis[g % 2]:
            d.wait()
        gi, gj = gs.pop(g)
        gi.wait()
        gj.wait()
        sa, si = _compute(g, (sa, si))
        scts[g] = _issue_scatter(g)
        if g >= 1:
            di, dj = scts.pop(g - 1)
            di.wait()
            dj.wait()
        if g + 2 < NCHUNK:
            dcs[g % 2], dis[g % 2] = _issue_in(g + 2)
    di, dj = scts.pop(NCHUNK - 1)
    di.wait()
    dj.wait()
    seA[...] = sa
    seI[...] = si
    pltpu.sync_copy(seA, sums_out.at[pl.ds(wid * 16, 16)])
    pltpu.sync_copy(seI, sums_out.at[pl.ds((NW + wid) * 16, 16)])
    plsc.subcore_barrier()
    pltpu.sync_copy(accS.at[pl.ds(s * RPT, RPT)],
                    acc_out.at[pl.ds(c * NP + s * RPT, RPT)])


@functools.partial(
    pl.kernel,
    out_type=jax.ShapeDtypeStruct((NW * 5 * 16,), F32),
    mesh=_mesh,
    compiler_params=_params,
    scratch_types=[
        pltpu.VMEM((NPW, 8), F32),
        pltpu.VMEM((NPW, 8), F32),
        pltpu.VMEM((NPW, 3), F32),
        pltpu.VMEM((NPW,), F32),
        pltpu.VMEM((NPW,), F32),
        pltpu.VMEM((NPW * 8,), F32),
        pltpu.VMEM((80,), F32),
    ],
)
def _k2(acc, llp, bcd, bcr, predf,
        part_out,
        a0, a1, bll, bbd, bbr, pbuf, outbuf):
    c = lax.axis_index("c")
    s = lax.axis_index("s")
    wid = s * NC + c
    nb = wid * NPW
    pltpu.sync_copy(acc.at[pl.ds(nb, NPW)], a0)
    pltpu.sync_copy(acc.at[pl.ds(NP + nb, NPW)], a1)
    pltpu.sync_copy(llp.at[pl.ds(nb, NPW)], bll)
    pltpu.sync_copy(bcd.at[pl.ds(nb, NPW)], bbd)
    pltpu.sync_copy(bcr.at[pl.ds(nb, NPW)], bbr)
    pltpu.sync_copy(predf.at[pl.ds(nb * 8, NPW * 8)], pbuf)

    def _node(t, carry):
        sf, sm, nd, nr = carry
        o = t * 16
        e = o + _iota16()
        fx = (plsc.load_gather(a0, [e, _col(0)]) + plsc.load_gather(a1, [e, _col(0)])
              + plsc.load_gather(bll, [e, _col(0)]))
        fy = (plsc.load_gather(a0, [e, _col(1)]) + plsc.load_gather(a1, [e, _col(1)])
              + plsc.load_gather(bll, [e, _col(1)]))
        fz = (plsc.load_gather(a0, [e, _col(2)]) + plsc.load_gather(a1, [e, _col(2)])
              + plsc.load_gather(bll, [e, _col(2)]))
        mx = plsc.load_gather(a0, [e, _col(3)]) + plsc.load_gather(a1, [e, _col(3)])
        my = plsc.load_gather(a0, [e, _col(4)]) + plsc.load_gather(a1, [e, _col(4)])
        mz = plsc.load_gather(a0, [e, _col(5)]) + plsc.load_gather(a1, [e, _col(5)])
        one = jnp.ones((16,), F32)
        zeroes = jnp.zeros((16,), F32)
        fd = jnp.where(bbd[pl.ds(o, 16)] < 0.5, one, zeroes)
        fr = jnp.where(bbr[pl.ds(o, 16)] < 0.5, one, zeroes)
        sf = sf + fd * (fx * fx + fy * fy + fz * fz)
        sm = sm + fr * (mx * mx + my * my + mz * mz)
        return sf, sm, nd + fd, nr + fr

    zero = jnp.zeros((16,), F32)
    sf, sm, nd, nr = lax.fori_loop(0, NPW // 16, _node, (zero, zero, zero, zero))

    def _p2(t, p2):
        v = pbuf[pl.ds(t * 16, 16)]
        return p2 + v * v

    p2 = lax.fori_loop(0, (NPW * 8) // 16, _p2, zero)
    outbuf[pl.ds(0, 16)] = sf
    outbuf[pl.ds(16, 16)] = sm
    outbuf[pl.ds(32, 16)] = nd
    outbuf[pl.ds(48, 16)] = nr
    outbuf[pl.ds(64, 16)] = p2
    pltpu.sync_copy(outbuf, part_out.at[pl.ds(wid * 80, 80)])


@functools.partial(
    pl.kernel,
    out_type=jax.ShapeDtypeStruct((16,), F32),
    mesh=_mesh,
    compiler_params=_params,
    scratch_types=[
        pltpu.VMEM((NW * 5 * 16,), F32),
        pltpu.VMEM((2 * NW * 16,), F32),
        pltpu.VMEM((16,), F32),
    ],
)
def _k3(part, sums, lout, pbuf, sbuf, obuf):
    c = lax.axis_index("c")
    s = lax.axis_index("s")
    wid = s * NC + c

    @pl.when(wid == 0)
    def _():
        pltpu.sync_copy(part, pbuf)
        pltpu.sync_copy(sums, sbuf)
        zero = jnp.zeros((16,), F32)

        def _acc5(w, carry):
            sf, sm, nd, nr, p2 = carry
            o = w * 80
            return (sf + pbuf[pl.ds(o, 16)],
                    sm + pbuf[pl.ds(o + 16, 16)],
                    nd + pbuf[pl.ds(o + 32, 16)],
                    nr + pbuf[pl.ds(o + 48, 16)],
                    p2 + pbuf[pl.ds(o + 64, 16)])

        sf, sm, nd, nr, p2 = lax.fori_loop(0, NW, _acc5,
                                           (zero, zero, zero, zero, zero))

        def _acc2(w, carry):
            ea, eil = carry
            return (ea + sbuf[pl.ds(w * 16, 16)],
                    eil + sbuf[pl.ds((NW + w) * 16, 16)])

        ea, eil = lax.fori_loop(0, NW, _acc2, (zero, zero))
        sf_v = jnp.full((16,), jnp.sum(sf), F32)
        sm_v = jnp.full((16,), jnp.sum(sm), F32)
        nd_v = jnp.full((16,), jnp.sum(nd), F32)
        nr_v = jnp.full((16,), jnp.sum(nr), F32)
        p2_v = jnp.full((16,), jnp.sum(p2), F32)
        ea_v = jnp.maximum(jnp.full((16,), jnp.sum(ea) * (1.0 / N_ELEM), F32), 1.0)
        eil_v = jnp.maximum(jnp.full((16,), jnp.sum(eil) * (1.0 / N_ELEM), F32), 1.0)
        rf = _rsqrt(ea_v * ea_v * (3.0 * nd_v))
        rm = _rsqrt(eil_v * eil_v * (3.0 * nr_v))
        lossv = (sf_v * rf * rf + sm_v * rm * rm
                 + 1e-4 * (p2_v * (1.0 / (N_NODES * 6.0))))
        obuf[...] = lossv
        pltpu.sync_copy(obuf, lout)


def kernel(pred, connectivity, elem_directions, elem_lengths, prop_E, prop_A,
           prop_I22, line_load, bc_disp, bc_rot):
    npad = NP - N_NODES
    pred8 = jnp.pad(pred, ((0, npad), (0, 2)))
    llp = jnp.pad(line_load, ((0, npad), (0, 0)))
    bcd = jnp.pad(bc_disp[:, 0], (0, npad), constant_values=1.0)
    bcr = jnp.pad(bc_rot[:, 0], (0, npad), constant_values=1.0)

    cflat = jnp.stack([connectivity[:, 0], connectivity[:, 1]], 1).reshape(-1)
    dflat = jnp.stack([elem_directions[:, 0], elem_directions[:, 1],
                       elem_directions[:, 2]], 1).reshape(-1)
    acc, sums = _k1(cflat, dflat,
                    elem_lengths, prop_E, prop_A, prop_I22, pred8)
    part = _k2(acc, llp, bcd, bcr, pred8.reshape(-1))
    lout = _k3(part, sums)
    return lout[0]


# speedup vs baseline: 11.7172x; 11.7172x over previous
import functools

import jax
import jax.numpy as jnp
from jax import lax
from jax.experimental import pallas as pl
from jax.experimental.pallas import tpu as pltpu
from jax.experimental.pallas import tpu_sc as plsc

N_NODES = 50000
N_ELEM = 800000

NC, NS, LANES = 2, 16, 16
NW = NC * NS
NP = 50176
EPW = N_ELEM // NW
K = 1000
KB = 1008
NSTEP = KB // 16
NCHUNK = EPW // K
RPT = NP // NS
ZR = RPT // 8
NPW = NP // NW
F32 = jnp.float32
I32 = jnp.int32

_mesh = plsc.VectorSubcoreMesh(core_axis_name="c", subcore_axis_name="s")
_params = pltpu.CompilerParams(needs_layout_passes=False,
                               use_tc_tiling_on_sc=False)


def _rsqrt(x):
    x = jnp.maximum(x, 1e-16)
    i = plsc.bitcast(x, I32)
    i = jnp.int32(0x5F3759DF) - (i >> 1)
    y = plsc.bitcast(i, F32)
    for _ in range(2):
        y = y * (1.5 - 0.5 * x * y * y)
    return y


def _iota16():
    return lax.iota(I32, 16)


def _col(c):
    return jnp.full((16,), c, I32)


@functools.partial(
    pl.kernel,
    out_type=[
        jax.ShapeDtypeStruct((2 * NP, 8), F32),
        jax.ShapeDtypeStruct((2 * NW * 16,), F32),
    ],
    mesh=_mesh,
    compiler_params=_params,
    scratch_types=[
        pltpu.VMEM((3, K), I32),
        pltpu.VMEM((3, K), I32),
        pltpu.VMEM((2, KB, 8), F32),
        pltpu.VMEM((2, KB, 8), F32),
        pltpu.VMEM((2, KB, 8), F32),
        pltpu.VMEM((2, KB, 8), F32),
        pltpu.VMEM((2, 7, KB), F32),
        pltpu.VMEM((ZR, 8), F32),
        pltpu.VMEM((16,), F32),
        pltpu.VMEM((16,), F32),
        pltpu.VMEM_SHARED((NP, 8), F32),
        [pltpu.SemaphoreType.DMA] * 3,
        [pltpu.SemaphoreType.DMA] * 3,
        [pltpu.SemaphoreType.DMA] * 2,
        [pltpu.SemaphoreType.DMA] * 2,
        [pltpu.SemaphoreType.DMA] * 2,
        [pltpu.SemaphoreType.DMA] * 2,
        [pltpu.SemaphoreType.DMA] * 2,
    ],
)
def _k1(ci, cj, xs, ys, zs, lh, eh, ah, ih, pred8,
        acc_out, sums_out,
        cib, cjb, rows_i, rows_j, srow_i, srow_j, pbuf,
        zbuf, seA, seI, accS,
        sem_ci, sem_cj, sem_p, sem_gi, sem_gj, sem_si, sem_sj):
    c = lax.axis_index("c")
    s = lax.axis_index("s")
    wid = s * NC + c

    def _zb1(t, carry):
        e = t * 16 + _iota16()
        z = jnp.zeros((16,), F32)
        plsc.store_scatter(zbuf, [e >> 3, e & 7], z)
        return carry

    lax.fori_loop(0, (ZR * 8) // 16, _zb1, 0)
    for q in range(8):
        pltpu.sync_copy(zbuf, accS.at[pl.ds(s * RPT + q * ZR, ZR)])

    def _zpad(t, carry):
        e = t * 16 + _iota16()
        z = jnp.zeros((16,), F32)
        for b in range(2):
            plsc.store_scatter(srow_i.at[b], [e, _col(6)], z)
            plsc.store_scatter(srow_i.at[b], [e, _col(7)], z)
            plsc.store_scatter(srow_j.at[b], [e, _col(6)], z)
            plsc.store_scatter(srow_j.at[b], [e, _col(7)], z)
        return carry

    lax.fori_loop(0, NSTEP, _zpad, 0)
    plsc.subcore_barrier()

    ebase = wid * EPW

    def _issue_in(g):
        base = ebase + g * K
        d1 = pltpu.async_copy(ci.at[pl.ds(base, K)], cib.at[g % 3],
                              sem_ci[g % 3])
        d2 = pltpu.async_copy(cj.at[pl.ds(base, K)], cjb.at[g % 3],
                              sem_cj[g % 3])
        srcs = (xs, ys, zs, lh, eh, ah, ih)
        dps = tuple(
            pltpu.async_copy(srcs[r].at[pl.ds(base, K)],
                             pbuf.at[g % 2].at[r].at[pl.ds(0, K)],
                             sem_p[g % 2])
            for r in range(7))
        return (d1, d2), dps

    def _issue_gather(g):
        gi = pltpu.async_copy(pred8.at[cib.at[g % 3]],
                              rows_i.at[g % 2].at[pl.ds(0, K)], sem_gi[g % 2])
        gj = pltpu.async_copy(pred8.at[cjb.at[g % 3]],
                              rows_j.at[g % 2].at[pl.ds(0, K)], sem_gj[g % 2])
        return gi, gj

    def _issue_scatter(g):
        di = pltpu.async_copy(srow_i.at[g % 2].at[pl.ds(0, K)],
                              accS.at[cib.at[g % 3]], sem_si[g % 2], add=True)
        dj = pltpu.async_copy(srow_j.at[g % 2].at[pl.ds(0, K)],
                              accS.at[cjb.at[g % 3]], sem_sj[g % 2], add=True)
        return di, dj

    def _compute(g, carry):
        sa, si = carry
        pb = pbuf.at[g % 2]
        ri = rows_i.at[g % 2]
        rj = rows_j.at[g % 2]
        swi = srow_i.at[g % 2]
        swj = srow_j.at[g % 2]

        def _step(t, carry2):
            sa2, si2 = carry2
            o = t * 16
            e = o + _iota16()
            ax = pb[0, pl.ds(o, 16)]
            ay = pb[1, pl.ds(o, 16)]
            az = pb[2, pl.ds(o, 16)]
            lv = pb[3, pl.ds(o, 16)]
            ev = pb[4, pl.ds(o, 16)]
            av = pb[5, pl.ds(o, 16)]
            iv = pb[6, pl.ds(o, 16)]
            par = jnp.abs(ay) > 0.99
            zpx = jnp.where(par, ay, -az)
            zpy = jnp.where(par, -ax, jnp.zeros((16,), F32))
            zpz = jnp.where(par, jnp.zeros((16,), F32), ax)
            rs = _rsqrt(zpx * zpx + zpy * zpy + zpz * zpz)
            zx_, zy_, zz_ = zpx * rs, zpy * rs, zpz * rs
            ypx = zy_ * az - zz_ * ay
            ypy = zz_ * ax - zx_ * az
            ypz = zx_ * ay - zy_ * ax
            rs2 = _rsqrt(ypx * ypx + ypy * ypy + ypz * ypz)
            yx_, yy_, yz_ = ypx * rs2, ypy * rs2, ypz * rs2
            uix = plsc.load_gather(ri, [e, _col(0)])
            uiy = plsc.load_gather(ri, [e, _col(1)])
            uiz = plsc.load_gather(ri, [e, _col(2)])
            tix = plsc.load_gather(ri, [e, _col(3)])
            tiy = plsc.load_gather(ri, [e, _col(4)])
            tiz = plsc.load_gather(ri, [e, _col(5)])
            ujx = plsc.load_gather(rj, [e, _col(0)])
            ujy = plsc.load_gather(rj, [e, _col(1)])
            ujz = plsc.load_gather(rj, [e, _col(2)])
            tjx = plsc.load_gather(rj, [e, _col(3)])
            tjy = plsc.load_gather(rj, [e, _col(4)])
            tjz = plsc.load_gather(rj, [e, _col(5)])
            dux, duy, duz = ujx - uix, ujy - uiy, ujz - uiz
            irl = _rsqrt(lv)
            invl = irl * irl
            ei = ev * iv
            ei_l = ei * invl
            ei_l2 = ei_l * invl
            ei_l3 = ei_l2 * invl
            axial = dux * ax + duy * ay + duz * az
            na = ev * av * invl * axial
            dwz = dux * zx_ + duy * zy_ + duz * zz_
            thyi = tix * yx_ + tiy * yy_ + tiz * yz_
            thyj = tjx * yx_ + tjy * yy_ + tjz * yz_
            vz = 12.0 * ei_l3 * dwz - 6.0 * ei_l2 * (thyi + thyj)
            myi = 6.0 * ei_l2 * dwz - ei_l * (4.0 * thyi + 2.0 * thyj)
            myj = 6.0 * ei_l2 * dwz - ei_l * (2.0 * thyi + 4.0 * thyj)
            dwy = dux * yx_ + duy * yy_ + duz * yz_
            thzi = tix * zx_ + tiy * zy_ + tiz * zz_
            thzj = tjx * zx_ + tjy * zy_ + tjz * zz_
            vy = 12.0 * ei_l3 * dwy + 6.0 * ei_l2 * (thzi + thzj)
            mzi = -6.0 * ei_l2 * dwy - ei_l * (4.0 * thzi + 2.0 * thzj)
            mzj = -6.0 * ei_l2 * dwy - ei_l * (2.0 * thzi + 4.0 * thzj)
            fx = na * ax + vz * zx_ + vy * yx_
            fy = na * ay + vz * zy_ + vy * yy_
            fz = na * az + vz * zz_ + vy * yz_
            plsc.store_scatter(swi, [e, _col(0)], fx)
            plsc.store_scatter(swi, [e, _col(1)], fy)
            plsc.store_scatter(swi, [e, _col(2)], fz)
            plsc.store_scatter(swi, [e, _col(3)], myi * yx_ + mzi * zx_)
            plsc.store_scatter(swi, [e, _col(4)], myi * yy_ + mzi * zy_)
            plsc.store_scatter(swi, [e, _col(5)], myi * yz_ + mzi * zz_)
            plsc.store_scatter(swj, [e, _col(0)], -fx)
            plsc.store_scatter(swj, [e, _col(1)], -fy)
            plsc.store_scatter(swj, [e, _col(2)], -fz)
            plsc.store_scatter(swj, [e, _col(3)], myj * yx_ + mzj * zx_)
            plsc.store_scatter(swj, [e, _col(4)], myj * yy_ + mzj * zy_)
            plsc.store_scatter(swj, [e, _col(5)], myj * yz_ + mzj * zz_)
            valid = jnp.where(e < K, jnp.ones((16,), F32), jnp.zeros((16,), F32))
            return sa2 + valid * (ev * av), si2 + valid * ei_l

        return lax.fori_loop(0, NSTEP, _step, (sa, si))

    sa = jnp.zeros((16,), F32)
    si = jnp.zeros((16,), F32)
    dcs = [None, None, None]
    dis = [None, None]
    dcs[0], dis[0] = _issue_in(0)
    dcs[1], dis[1] = _issue_in(1)
    for d in dcs[0]:
        d.wait()
    gs = {0: _issue_gather(0)}
    scts = {}
    for g in range(NCHUNK):
        if g + 1 < NCHUNK:
            for d in dcs[(g + 1) % 3]:
                d.wait()
            gs[g + 1] = _issue_gather(g + 1)
        for d in dis[g % 2]:
            d.wait()
        gi, gj = gs.pop(g)
        gi.wait()
        gj.wait()
        sa, si = _compute(g, (sa, si))
        scts[g] = _issue_scatter(g)
        if g >= 1:
            di, dj = scts.pop(g - 1)
            di.wait()
            dj.wait()
        if g + 2 < NCHUNK:
            dcs[(g + 2) % 3], dis[g % 2] = _issue_in(g + 2)
    di, dj = scts.pop(NCHUNK - 1)
    di.wait()
    dj.wait()
    seA[...] = sa
    seI[...] = si
    pltpu.sync_copy(seA, sums_out.at[pl.ds(wid * 16, 16)])
    pltpu.sync_copy(seI, sums_out.at[pl.ds((NW + wid) * 16, 16)])
    plsc.subcore_barrier()
    pltpu.sync_copy(accS.at[pl.ds(s * RPT, RPT)],
                    acc_out.at[pl.ds(c * NP + s * RPT, RPT)])


@functools.partial(
    pl.kernel,
    out_type=jax.ShapeDtypeStruct((NW * 5 * 16,), F32),
    mesh=_mesh,
    compiler_params=_params,
    scratch_types=[
        pltpu.VMEM((NPW, 8), F32),
        pltpu.VMEM((NPW, 8), F32),
        pltpu.VMEM((NPW, 3), F32),
        pltpu.VMEM((NPW,), F32),
        pltpu.VMEM((NPW,), F32),
        pltpu.VMEM((NPW * 8,), F32),
        pltpu.VMEM((80,), F32),
    ],
)
def _k2(acc, llp, bcd, bcr, predf,
        part_out,
        a0, a1, bll, bbd, bbr, pbuf, outbuf):
    c = lax.axis_index("c")
    s = lax.axis_index("s")
    wid = s * NC + c
    nb = wid * NPW
    pltpu.sync_copy(acc.at[pl.ds(nb, NPW)], a0)
    pltpu.sync_copy(acc.at[pl.ds(NP + nb, NPW)], a1)
    pltpu.sync_copy(llp.at[pl.ds(nb, NPW)], bll)
    pltpu.sync_copy(bcd.at[pl.ds(nb, NPW)], bbd)
    pltpu.sync_copy(bcr.at[pl.ds(nb, NPW)], bbr)
    pltpu.sync_copy(predf.at[pl.ds(nb * 8, NPW * 8)], pbuf)

    def _node(t, carry):
        sf, sm, nd, nr = carry
        o = t * 16
        e = o + _iota16()
        fx = (plsc.load_gather(a0, [e, _col(0)]) + plsc.load_gather(a1, [e, _col(0)])
              + plsc.load_gather(bll, [e, _col(0)]))
        fy = (plsc.load_gather(a0, [e, _col(1)]) + plsc.load_gather(a1, [e, _col(1)])
              + plsc.load_gather(bll, [e, _col(1)]))
        fz = (plsc.load_gather(a0, [e, _col(2)]) + plsc.load_gather(a1, [e, _col(2)])
              + plsc.load_gather(bll, [e, _col(2)]))
        mx = plsc.load_gather(a0, [e, _col(3)]) + plsc.load_gather(a1, [e, _col(3)])
        my = plsc.load_gather(a0, [e, _col(4)]) + plsc.load_gather(a1, [e, _col(4)])
        mz = plsc.load_gather(a0, [e, _col(5)]) + plsc.load_gather(a1, [e, _col(5)])
        one = jnp.ones((16,), F32)
        zeroes = jnp.zeros((16,), F32)
        fd = jnp.where(bbd[pl.ds(o, 16)] < 0.5, one, zeroes)
        fr = jnp.where(bbr[pl.ds(o, 16)] < 0.5, one, zeroes)
        sf = sf + fd * (fx * fx + fy * fy + fz * fz)
        sm = sm + fr * (mx * mx + my * my + mz * mz)
        return sf, sm, nd + fd, nr + fr

    zero = jnp.zeros((16,), F32)
    sf, sm, nd, nr = lax.fori_loop(0, NPW // 16, _node, (zero, zero, zero, zero))

    def _p2(t, p2):
        v = pbuf[pl.ds(t * 16, 16)]
        return p2 + v * v

    p2 = lax.fori_loop(0, (NPW * 8) // 16, _p2, zero)
    outbuf[pl.ds(0, 16)] = sf
    outbuf[pl.ds(16, 16)] = sm
    outbuf[pl.ds(32, 16)] = nd
    outbuf[pl.ds(48, 16)] = nr
    outbuf[pl.ds(64, 16)] = p2
    pltpu.sync_copy(outbuf, part_out.at[pl.ds(wid * 80, 80)])


@functools.partial(
    pl.kernel,
    out_type=jax.ShapeDtypeStruct((16,), F32),
    mesh=_mesh,
    compiler_params=_params,
    scratch_types=[
        pltpu.VMEM((NW * 5 * 16,), F32),
        pltpu.VMEM((2 * NW * 16,), F32),
        pltpu.VMEM((16,), F32),
    ],
)
def _k3(part, sums, lout, pbuf, sbuf, obuf):
    c = lax.axis_index("c")
    s = lax.axis_index("s")
    wid = s * NC + c

    @pl.when(wid == 0)
    def _():
        pltpu.sync_copy(part, pbuf)
        pltpu.sync_copy(sums, sbuf)
        zero = jnp.zeros((16,), F32)

        def _acc5(w, carry):
            sf, sm, nd, nr, p2 = carry
            o = w * 80
            return (sf + pbuf[pl.ds(o, 16)],
                    sm + pbuf[pl.ds(o + 16, 16)],
                    nd + pbuf[pl.ds(o + 32, 16)],
                    nr + pbuf[pl.ds(o + 48, 16)],
                    p2 + pbuf[pl.ds(o + 64, 16)])

        sf, sm, nd, nr, p2 = lax.fori_loop(0, NW, _acc5,
                                           (zero, zero, zero, zero, zero))

        def _acc2(w, carry):
            ea, eil = carry
            return (ea + sbuf[pl.ds(w * 16, 16)],
                    eil + sbuf[pl.ds((NW + w) * 16, 16)])

        ea, eil = lax.fori_loop(0, NW, _acc2, (zero, zero))
        sf_v = jnp.full((16,), jnp.sum(sf), F32)
        sm_v = jnp.full((16,), jnp.sum(sm), F32)
        nd_v = jnp.full((16,), jnp.sum(nd), F32)
        nr_v = jnp.full((16,), jnp.sum(nr), F32)
        p2_v = jnp.full((16,), jnp.sum(p2), F32)
        ea_v = jnp.maximum(jnp.full((16,), jnp.sum(ea) * (1.0 / N_ELEM), F32), 1.0)
        eil_v = jnp.maximum(jnp.full((16,), jnp.sum(eil) * (1.0 / N_ELEM), F32), 1.0)
        rf = _rsqrt(ea_v * ea_v * (3.0 * nd_v))
        rm = _rsqrt(eil_v * eil_v * (3.0 * nr_v))
        lossv = (sf_v * rf * rf + sm_v * rm * rm
                 + 1e-4 * (p2_v * (1.0 / (N_NODES * 6.0))))
        obuf[...] = lossv
        pltpu.sync_copy(obuf, lout)


def kernel(pred, connectivity, elem_directions, elem_lengths, prop_E, prop_A,
           prop_I22, line_load, bc_disp, bc_rot):
    npad = NP - N_NODES
    pred8 = jnp.pad(pred, ((0, npad), (0, 2)))
    llp = jnp.pad(line_load, ((0, npad), (0, 0)))
    bcd = jnp.pad(bc_disp[:, 0], (0, npad), constant_values=1.0)
    bcr = jnp.pad(bc_rot[:, 0], (0, npad), constant_values=1.0)

    acc, sums = _k1(connectivity[:, 0], connectivity[:, 1],
                    elem_directions[:, 0], elem_directions[:, 1],
                    elem_directions[:, 2],
                    elem_lengths, prop_E, prop_A, prop_I22, pred8)
    part = _k2(acc, llp, bcd, bcr, pred8.reshape(-1))
    lout = _k3(part, sums)
    return lout[0]
